# baseline (speedup 1.0000x reference)
import functools
import math

import jax
import jax.numpy as jnp
from jax.experimental import pallas as pl
from jax.experimental.pallas import tpu as pltpu

_EPS = 0.05
_ITERS = 200


def _emd_kernel(pc1_ref, pc2_ref, pc1t_ref, pc2t_ref, out_ref, k_ref, kt_ref,
                *, n_pts, n_iters):
    eps = jnp.float32(_EPS)
    w = jnp.float32(1.0 / n_pts)

    def neg_scaled_dist(col_ref, row_ref):
        col = col_ref[0]
        row = row_ref[0]
        d2 = jnp.zeros((n_pts, n_pts), jnp.float32)
        for k in range(3):
            diff = col[:, k : k + 1] - row[k : k + 1, :]
            d2 = d2 + diff * diff
        return -jnp.sqrt(d2 + jnp.float32(1e-12)) / eps

    negc = neg_scaled_dist(pc1_ref, pc2t_ref)
    alpha = -jnp.max(negc, axis=1, keepdims=True)
    k_ref[...] = jnp.exp(negc + alpha)

    negct = neg_scaled_dist(pc2_ref, pc1t_ref)
    alphat = -jnp.max(negct, axis=0, keepdims=True)
    kt_ref[...] = jnp.exp(negct + alphat)

    floor = jnp.float32(1e-35)
    cap = jnp.float32(1e30)

    def body(_, carry):
        u, v = carry
        t1 = jnp.dot(k_ref[...], v)
        u = jnp.minimum(w / jnp.maximum(t1, floor), cap)
        t2 = jnp.dot(kt_ref[...], u)
        v = jnp.minimum(w / jnp.maximum(t2, floor), cap)
        return u, v

    u0 = jnp.full((n_pts, 128), w, jnp.float32)
    v0 = jnp.full((n_pts, 128), w, jnp.float32)
    u, v = jax.lax.fori_loop(0, n_iters, body, (u0, v0))

    kmat = k_ref[...]
    kc = kmat * (eps * (alpha - jnp.log(jnp.maximum(kmat, jnp.float32(1e-37)))))
    t3 = jnp.dot(kc, v)
    total = jnp.sum(u[:, :1] * t3[:, :1])
    out_ref[...] = jnp.full(out_ref.shape, total, jnp.float32)


def kernel(pc1, pc2):
    b, n, _ = pc1.shape
    pc1t = pc1.transpose(0, 2, 1)
    pc2t = pc2.transpose(0, 2, 1)
    per_batch = pl.pallas_call(
        functools.partial(_emd_kernel, n_pts=n, n_iters=_ITERS),
        grid=(b,),
        in_specs=[
            pl.BlockSpec((1, n, 3), lambda i: (i, 0, 0)),
            pl.BlockSpec((1, n, 3), lambda i: (i, 0, 0)),
            pl.BlockSpec((1, 3, n), lambda i: (i, 0, 0)),
            pl.BlockSpec((1, 3, n), lambda i: (i, 0, 0)),
        ],
        out_specs=pl.BlockSpec((1, 1, 128), lambda i: (i, 0, 0)),
        out_shape=jax.ShapeDtypeStruct((b, 1, 128), jnp.float32),
        scratch_shapes=[
            pltpu.VMEM((n, n), jnp.float32),
            pltpu.VMEM((n, n), jnp.float32),
        ],
        compiler_params=pltpu.CompilerParams(
            dimension_semantics=("parallel",),
            vmem_limit_bytes=100 * 1024 * 1024,
        ),
        name="sinkhorn_emd",
    )(pc1, pc2, pc1t, pc2t)
    return jnp.sum(per_batch[:, 0, 0])


# speedup vs baseline: 2.1229x; 2.1229x over previous
import functools
import math

import jax
import jax.numpy as jnp
from jax.experimental import pallas as pl
from jax.experimental.pallas import tpu as pltpu

_EPS = 0.05
_ITERS = 200


def _emd_kernel(pc1_ref, pc2t_ref, out_ref, negc2_ref, *, n_pts, n_iters):
    c2 = jnp.float32(math.log2(math.e) / _EPS)
    log2_w = jnp.float32(-math.log2(n_pts))

    a = pc1_ref[0]
    bt = pc2t_ref[0]

    d2 = jnp.zeros((n_pts, n_pts), jnp.float32)
    for k in range(3):
        diff = a[:, k : k + 1] - bt[k : k + 1, :]
        d2 = d2 + diff * diff
    negc2_ref[...] = -jnp.sqrt(d2 + jnp.float32(1e-12)) * c2

    tiny = jnp.float32(1e-30)
    ones_mxu = jnp.ones((n_pts, 128), jnp.float32)

    def body(_, carry):
        phi, gamma = carry
        e1 = jnp.exp2(negc2_ref[...] + (log2_w + gamma) + phi)
        s1 = jnp.maximum(jnp.dot(e1, ones_mxu)[:, :1], tiny)
        s2 = jnp.maximum(
            jnp.sum(e1 * (1.0 / s1), axis=0, keepdims=True), tiny)
        phi = phi - jnp.log2(s1)
        gamma = gamma - jnp.log2(s2)
        return phi, gamma

    phi0 = jnp.zeros((n_pts, 1), jnp.float32)
    gamma0 = jnp.zeros((1, n_pts), jnp.float32)
    phi, gamma = jax.lax.fori_loop(0, n_iters, body, (phi0, gamma0))

    negc2 = negc2_ref[...]
    p = jnp.exp2(2.0 * log2_w + phi + gamma + negc2)
    total = jnp.float32(-_EPS * math.log(2.0)) * jnp.sum(p * negc2)
    out_ref[...] = jnp.full(out_ref.shape, total, jnp.float32)


def kernel(pc1, pc2):
    b, n, _ = pc1.shape
    pc2t = pc2.transpose(0, 2, 1)
    per_batch = pl.pallas_call(
        functools.partial(_emd_kernel, n_pts=n, n_iters=_ITERS),
        grid=(b,),
        in_specs=[
            pl.BlockSpec((1, n, 3), lambda i: (i, 0, 0)),
            pl.BlockSpec((1, 3, n), lambda i: (i, 0, 0)),
        ],
        out_specs=pl.BlockSpec((1, 1, 128), lambda i: (i, 0, 0)),
        out_shape=jax.ShapeDtypeStruct((b, 1, 128), jnp.float32),
        scratch_shapes=[pltpu.VMEM((n, n), jnp.float32)],
        compiler_params=pltpu.CompilerParams(
            dimension_semantics=("parallel",),
            vmem_limit_bytes=100 * 1024 * 1024,
        ),
        name="sinkhorn_emd",
    )(pc1, pc2t)
    return jnp.sum(per_batch[:, 0, 0])


# speedup vs baseline: 2.1472x; 1.0114x over previous
import functools
import math

import jax
import jax.numpy as jnp
from jax.experimental import pallas as pl
from jax.experimental.pallas import tpu as pltpu

_EPS = 0.05
_ITERS = 200


def _emd_kernel(pc1_ref, pc2t_ref, out_ref, negc2_ref, *, n_pts, n_iters):
    c2 = jnp.float32(math.log2(math.e) / _EPS)
    log2_w = jnp.float32(-math.log2(n_pts))

    a = pc1_ref[0]
    bt = pc2t_ref[0]

    d2 = jnp.zeros((n_pts, n_pts), jnp.float32)
    for k in range(3):
        diff = a[:, k : k + 1] - bt[k : k + 1, :]
        d2 = d2 + diff * diff
    negc2_ref[...] = -jnp.sqrt(d2 + jnp.float32(1e-12)) * c2

    tiny = jnp.float32(1e-30)
    ones_mxu = jnp.ones((n_pts, 128), jnp.float32)
    n_chunks = 4
    rows = n_pts // n_chunks

    def body(_, carry):
        phi, gamma = carry
        t_row = log2_w + gamma
        phi_new = []
        s2 = jnp.zeros((1, n_pts), jnp.float32)
        for c in range(n_chunks):
            sl = slice(c * rows, (c + 1) * rows)
            e1 = jnp.exp2(negc2_ref[sl, :] + t_row + phi[sl, :])
            s1 = jnp.maximum(jnp.dot(e1, ones_mxu)[:, :1], tiny)
            s2 = s2 + jnp.sum(e1 * (1.0 / s1), axis=0, keepdims=True)
            phi_new.append(phi[sl, :] - jnp.log2(s1))
        phi = jnp.concatenate(phi_new, axis=0)
        gamma = gamma - jnp.log2(jnp.maximum(s2, tiny))
        return phi, gamma

    phi0 = jnp.zeros((n_pts, 1), jnp.float32)
    gamma0 = jnp.zeros((1, n_pts), jnp.float32)
    phi, gamma = jax.lax.fori_loop(0, n_iters, body, (phi0, gamma0))

    negc2 = negc2_ref[...]
    p = jnp.exp2(2.0 * log2_w + phi + gamma + negc2)
    total = jnp.float32(-_EPS * math.log(2.0)) * jnp.sum(p * negc2)
    out_ref[...] = jnp.full(out_ref.shape, total, jnp.float32)


def kernel(pc1, pc2):
    b, n, _ = pc1.shape
    pc2t = pc2.transpose(0, 2, 1)
    per_batch = pl.pallas_call(
        functools.partial(_emd_kernel, n_pts=n, n_iters=_ITERS),
        grid=(b,),
        in_specs=[
            pl.BlockSpec((1, n, 3), lambda i: (i, 0, 0)),
            pl.BlockSpec((1, 3, n), lambda i: (i, 0, 0)),
        ],
        out_specs=pl.BlockSpec((1, 1, 128), lambda i: (i, 0, 0)),
        out_shape=jax.ShapeDtypeStruct((b, 1, 128), jnp.float32),
        scratch_shapes=[pltpu.VMEM((n, n), jnp.float32)],
        compiler_params=pltpu.CompilerParams(
            dimension_semantics=("parallel",),
            vmem_limit_bytes=100 * 1024 * 1024,
        ),
        name="sinkhorn_emd",
    )(pc1, pc2t)
    return jnp.sum(per_batch[:, 0, 0])


# speedup vs baseline: 3.5024x; 1.6312x over previous
import functools
import math

import jax
import jax.numpy as jnp
from jax.experimental import pallas as pl
from jax.experimental.pallas import tpu as pltpu

_EPS = 0.05
_ITERS = 200


def _emd_kernel(pc1_ref, pc2t_ref, out_ref, k_ref, *, n_pts, n_iters):
    eps = jnp.float32(_EPS)
    w = jnp.float32(1.0 / n_pts)

    a = pc1_ref[0]
    bt = pc2t_ref[0]

    d2 = jnp.zeros((n_pts, n_pts), jnp.float32)
    for k in range(3):
        diff = a[:, k : k + 1] - bt[k : k + 1, :]
        d2 = d2 + diff * diff
    negc = -jnp.sqrt(d2 + jnp.float32(1e-12)) / eps

    alpha = -jnp.max(negc, axis=1, keepdims=True)
    k_ref[...] = jnp.exp(negc + alpha)

    floor = jnp.float32(1e-35)
    cap = jnp.float32(1e30)

    def body(_, carry):
        u, v = carry
        kmat = k_ref[...]
        t1 = jnp.sum(kmat * v, axis=1, keepdims=True)
        u = jnp.minimum(w / jnp.maximum(t1, floor), cap)
        t2 = jnp.sum(kmat * u, axis=0, keepdims=True)
        v = jnp.minimum(w / jnp.maximum(t2, floor), cap)
        return u, v

    u0 = jnp.full((n_pts, 1), w, jnp.float32)
    v0 = jnp.full((1, n_pts), w, jnp.float32)
    u, v = jax.lax.fori_loop(0, n_iters, body, (u0, v0))

    kmat = k_ref[...]
    kc = kmat * (eps * (alpha - jnp.log(jnp.maximum(kmat, jnp.float32(1e-37)))))
    total = jnp.sum(u * jnp.sum(kc * v, axis=1, keepdims=True))
    out_ref[...] = jnp.full(out_ref.shape, total, jnp.float32)


def kernel(pc1, pc2):
    b, n, _ = pc1.shape
    pc2t = pc2.transpose(0, 2, 1)
    per_batch = pl.pallas_call(
        functools.partial(_emd_kernel, n_pts=n, n_iters=_ITERS),
        grid=(b,),
        in_specs=[
            pl.BlockSpec((1, n, 3), lambda i: (i, 0, 0)),
            pl.BlockSpec((1, 3, n), lambda i: (i, 0, 0)),
        ],
        out_specs=pl.BlockSpec((1, 1, 128), lambda i: (i, 0, 0)),
        out_shape=jax.ShapeDtypeStruct((b, 1, 128), jnp.float32),
        scratch_shapes=[pltpu.VMEM((n, n), jnp.float32)],
        compiler_params=pltpu.CompilerParams(
            dimension_semantics=("parallel",),
            vmem_limit_bytes=100 * 1024 * 1024,
        ),
        name="sinkhorn_emd",
    )(pc1, pc2t)
    return jnp.sum(per_batch[:, 0, 0])
